# Initial kernel scaffold; baseline (speedup 1.0000x reference)
import jax
import jax.numpy as jnp
from jax import lax
from jax.experimental import pallas as pl
from jax.experimental.pallas import tpu as pltpu
from jax.experimental.pallas import tpu_sc as plsc

N = 10000
E = 320000
DIM = 128
H = 8
DK = 16
ED = 16

NC = 2
NS = 16
NW = NC * NS
EPW = E // NW
CE = 80
NCHUNK = EPW // CE
GP = CE // 16
RPT = N // NS


def _tables_body(x_ref, wq_ref, bq_ref, wkv_ref, bkv_ref, q_ref, kv_ref):
    xb = x_ref[...]
    q_ref[...] = lax.dot_general(xb, wq_ref[...], (((1,), (1,)), ((), ()))) + bq_ref[...]
    kv_ref[...] = lax.dot_general(xb, wkv_ref[...], (((1,), (1,)), ((), ()))) + bkv_ref[...]


def _tables(x, wq, bq, wkv, bkv):
    bn = 1000
    return pl.pallas_call(
        _tables_body,
        grid=(N // bn,),
        in_specs=[
            pl.BlockSpec((bn, DIM), lambda i: (i, 0)),
            pl.BlockSpec((DIM, DIM), lambda i: (0, 0)),
            pl.BlockSpec((1, DIM), lambda i: (0, 0)),
            pl.BlockSpec((2 * DIM, DIM), lambda i: (0, 0)),
            pl.BlockSpec((1, 2 * DIM), lambda i: (0, 0)),
        ],
        out_specs=[
            pl.BlockSpec((bn, DIM), lambda i: (i, 0)),
            pl.BlockSpec((bn, 2 * DIM), lambda i: (i, 0)),
        ],
        out_shape=[
            jax.ShapeDtypeStruct((N, DIM), jnp.float32),
            jax.ShapeDtypeStruct((N, 2 * DIM), jnp.float32),
        ],
    )(x, wq, bq, wkv, bkv)


def _ebias_body(ea_ref, w1_ref, b1_ref, w2_ref, b2_ref, o_ref):
    ea = ea_ref[...]
    h1 = lax.dot_general(ea, w1_ref[...], (((1,), (1,)), ((), ()))) + b1_ref[...]
    h1 = h1 * jax.nn.sigmoid(h1)
    o_ref[...] = lax.dot_general(h1, w2_ref[...], (((1,), (1,)), ((), ()))) + b2_ref[...]


def _edge_bias(ea, w1, b1, w2p, b2p):
    be = 20000
    return pl.pallas_call(
        _ebias_body,
        grid=(E // be,),
        in_specs=[
            pl.BlockSpec((be, ED), lambda i: (i, 0)),
            pl.BlockSpec((ED, ED), lambda i: (0, 0)),
            pl.BlockSpec((1, ED), lambda i: (0, 0)),
            pl.BlockSpec((16, ED), lambda i: (0, 0)),
            pl.BlockSpec((1, 16), lambda i: (0, 0)),
        ],
        out_specs=pl.BlockSpec((be, 16), lambda i: (i, 0)),
        out_shape=jax.ShapeDtypeStruct((E, 16), jnp.float32),
    )(ea, w1, b1, w2p, b2p)


def _sc_body(qt, kvt, ei, bias, num_out, den_out,
             ii_v, jj_v, qrows, kvrows, bias_v, sbuf, wbuf, zbuf, zden,
             num_sh, den_sh, sem):
    c = lax.axis_index("c")
    s = lax.axis_index("s")
    wid = c * NS + s
    z16 = jnp.zeros((16,), jnp.float32)
    iota16 = lax.iota(jnp.int32, 16)

    def _z128(i, _):
        for cc in range(8):
            zbuf[i, pl.ds(cc * 16, 16)] = z16
        return 0
    lax.fori_loop(0, 125, _z128, 0)

    def _z16(i, _):
        zden[i, :] = z16
        return 0
    lax.fori_loop(0, RPT, _z16, 0)

    def _zs(i, _):
        sbuf[i, :] = z16
        return 0
    lax.fori_loop(0, CE, _zs, 0)

    for k in range(5):
        pltpu.sync_copy(zbuf, num_sh.at[pl.ds(s * RPT + k * 125, 125)])
    pltpu.sync_copy(zden, den_sh.at[pl.ds(s * RPT, RPT)])
    plsc.subcore_barrier()

    def chunk_body(ch, _):
        base = wid * EPW + ch * CE
        d1 = pltpu.async_copy(ei.at[0, pl.ds(base, CE)], ii_v, sem)
        d2 = pltpu.async_copy(ei.at[1, pl.ds(base, CE)], jj_v, sem)
        d3 = pltpu.async_copy(bias.at[pl.ds(base, CE)], bias_v, sem)
        d1.wait()
        d2.wait()
        d3.wait()
        g1 = pltpu.async_copy(qt.at[jj_v], qrows, sem)
        g2 = pltpu.async_copy(kvt.at[ii_v], kvrows, sem)
        g1.wait()
        g2.wait()

        def group_body(g, _):
            rows = g * 16 + iota16
            for h in range(8):
                hcol = jnp.full((16,), h, jnp.int32)
                bh = plsc.load_gather(bias_v, [rows, hcol])
                dot = z16
                for d in range(16):
                    cc = jnp.full((16,), h * DK + d, jnp.int32)
                    qv = plsc.load_gather(qrows, [rows, cc])
                    kv = plsc.load_gather(kvrows, [rows, cc])
                    dot = dot + qv * kv
                sh = jnp.exp(dot * 0.25 + bh)
                plsc.store_scatter(sbuf, [rows, hcol], sh)
            for e in range(16):
                row = g * 16 + e
                for h in range(8):
                    sv = sbuf[row, h]
                    vv = kvrows[row, pl.ds(DIM + h * DK, DK)]
                    wbuf[row, pl.ds(h * DK, DK)] = vv * sv
            return 0

        lax.fori_loop(0, GP, group_body, 0)
        pltpu.sync_copy(wbuf, num_sh.at[jj_v], add=True)
        pltpu.sync_copy(sbuf, den_sh.at[jj_v], add=True)
        return 0

    lax.fori_loop(0, NCHUNK, chunk_body, 0)
    plsc.subcore_barrier()

    for k in range(5):
        pltpu.sync_copy(num_sh.at[pl.ds(s * RPT + k * 125, 125)], zbuf)
        pltpu.sync_copy(zbuf, num_out.at[c, pl.ds(s * RPT + k * 125, 125)])
    pltpu.sync_copy(den_sh.at[pl.ds(s * RPT, RPT)], zden)
    pltpu.sync_copy(zden, den_out.at[c, pl.ds(s * RPT, RPT)])


def _sc_attn(qt, kvt, ei, bias):
    mesh = plsc.VectorSubcoreMesh(core_axis_name="c", subcore_axis_name="s")
    return pl.kernel(
        _sc_body,
        out_type=[
            jax.ShapeDtypeStruct((NC, N, DIM), jnp.float32),
            jax.ShapeDtypeStruct((NC, N, 16), jnp.float32),
        ],
        mesh=mesh,
        scratch_types=[
            pltpu.VMEM((CE,), jnp.int32),
            pltpu.VMEM((CE,), jnp.int32),
            pltpu.VMEM((CE, DIM), jnp.float32),
            pltpu.VMEM((CE, 2 * DIM), jnp.float32),
            pltpu.VMEM((CE, 16), jnp.float32),
            pltpu.VMEM((CE, 16), jnp.float32),
            pltpu.VMEM((CE, DIM), jnp.float32),
            pltpu.VMEM((125, DIM), jnp.float32),
            pltpu.VMEM((RPT, 16), jnp.float32),
            pltpu.VMEM_SHARED((N, DIM), jnp.float32),
            pltpu.VMEM_SHARED((N, 16), jnp.float32),
            pltpu.SemaphoreType.DMA,
        ],
    )(qt, kvt, ei, bias)


def _fin_body(num_ref, den_ref, wo_ref, bo_ref, o_ref):
    nsum = num_ref[0] + num_ref[1]
    dsum = den_ref[0] + den_ref[1]
    r = lax.broadcasted_iota(jnp.int32, (16, DIM), 0)
    ccol = lax.broadcasted_iota(jnp.int32, (16, DIM), 1)
    sel = (ccol // DK == r).astype(jnp.float32)
    den128 = lax.dot_general(dsum, sel, (((1,), (0,)), ((), ())))
    attn = nsum / (den128 + 1e-12)
    o_ref[...] = lax.dot_general(attn, wo_ref[...], (((1,), (1,)), ((), ()))) + bo_ref[...]


def _finalize(num_p, den_p, wo, bo):
    bn = 1000
    return pl.pallas_call(
        _fin_body,
        grid=(N // bn,),
        in_specs=[
            pl.BlockSpec((NC, bn, DIM), lambda i: (0, i, 0)),
            pl.BlockSpec((NC, bn, 16), lambda i: (0, i, 0)),
            pl.BlockSpec((DIM, DIM), lambda i: (0, 0)),
            pl.BlockSpec((1, DIM), lambda i: (0, 0)),
        ],
        out_specs=pl.BlockSpec((bn, DIM), lambda i: (i, 0)),
        out_shape=jax.ShapeDtypeStruct((N, DIM), jnp.float32),
    )(num_p, den_p, wo, bo)


def kernel(x, edge_index, edge_attr, W_Q, b_Q, W_K, b_K, W_V, b_V, W_O, b_O,
           eb_W1, eb_b1, eb_W2, eb_b2):
    ei = edge_index.astype(jnp.int32)
    wkv = jnp.concatenate([W_K, W_V], axis=0)
    bkv = jnp.concatenate([b_K, b_V])[None, :]
    qt, kvt = _tables(x, W_Q, b_Q[None, :], wkv, bkv)
    w2p = jnp.zeros((16, ED), jnp.float32).at[:H].set(eb_W2)
    b2p = jnp.zeros((16,), jnp.float32).at[:H].set(eb_b2)
    ebias = _edge_bias(edge_attr, eb_W1, eb_b1[None, :], w2p, b2p[None, :])
    num_p, den_p = _sc_attn(qt, kvt, ei, ebias)
    return _finalize(num_p, den_p, W_O, b_O[None, :])


# speedup vs baseline: 3.7239x; 3.7239x over previous
import jax
import jax.numpy as jnp
from jax import lax
from jax.experimental import pallas as pl
from jax.experimental.pallas import tpu as pltpu
from jax.experimental.pallas import tpu_sc as plsc

N = 10000
E = 320000
DIM = 128
H = 8
DK = 16
ED = 16

NC = 2
NS = 16
NW = NC * NS
CE = 80
GP = CE // 16
NPAD = 10240
RPT = NPAD // NS


def _tables_body(x_ref, wq_ref, bq_ref, wk_ref, bk_ref, wv_ref, bv_ref,
                 q_ref, k_ref, v_ref):
    xb = x_ref[...]
    dn = (((1,), (1,)), ((), ()))
    q_ref[...] = lax.dot_general(xb, wq_ref[...], dn) + bq_ref[...]
    k_ref[...] = lax.dot_general(xb, wk_ref[...], dn) + bk_ref[...]
    v_ref[...] = lax.dot_general(xb, wv_ref[...], dn) + bv_ref[...]


def _tables(x, wq, bq, wk, bk, wv, bv):
    bn = 1000
    mspec = pl.BlockSpec((DIM, DIM), lambda i: (0, 0))
    bspec = pl.BlockSpec((1, DIM), lambda i: (0, 0))
    nspec = pl.BlockSpec((bn, DIM), lambda i: (i, 0))
    return pl.pallas_call(
        _tables_body,
        grid=(N // bn,),
        in_specs=[nspec, mspec, bspec, mspec, bspec, mspec, bspec],
        out_specs=[nspec, nspec, nspec],
        out_shape=[jax.ShapeDtypeStruct((N, DIM), jnp.float32)] * 3,
    )(x, wq, bq, wk, bk, wv, bv)


def _ebias_body(ea_ref, w1_ref, b1_ref, w2_ref, b2_ref, o_ref):
    ea = ea_ref[...]
    dn = (((1,), (1,)), ((), ()))
    h1 = lax.dot_general(ea, w1_ref[...], dn) + b1_ref[...]
    h1 = h1 * jax.nn.sigmoid(h1)
    o_ref[...] = lax.dot_general(h1, w2_ref[...], dn) + b2_ref[...]


def _edge_bias(ea, w1, b1, w2p, b2p):
    be = 20000
    return pl.pallas_call(
        _ebias_body,
        grid=(E // be,),
        in_specs=[
            pl.BlockSpec((be, ED), lambda i: (i, 0)),
            pl.BlockSpec((ED, ED), lambda i: (0, 0)),
            pl.BlockSpec((1, ED), lambda i: (0, 0)),
            pl.BlockSpec((16, ED), lambda i: (0, 0)),
            pl.BlockSpec((1, 16), lambda i: (0, 0)),
        ],
        out_specs=pl.BlockSpec((be, 16), lambda i: (i, 0)),
        out_shape=jax.ShapeDtypeStruct((E, 16), jnp.float32),
    )(ea, w1, b1, w2p, b2p)


HPC = H // NC
EPT = E // NS
NCHUNK = EPT // CE


def _sc_body(qt, kt, vt, src_idx, dst_idx, bias, num_out, den_out,
             ii_v, jj_v, qrows, krows, vrows, bias_v, sbuf, wbuf, zbuf, zden,
             num_sh, den_sh, sem):
    c = lax.axis_index("c")
    s = lax.axis_index("s")
    hoff = c * HPC
    z16 = jnp.zeros((16,), jnp.float32)
    iota16 = lax.iota(jnp.int32, 16)

    def _z64(i, _):
        for cc in range(4):
            zbuf[i, pl.ds(cc * 16, 16)] = z16
        return 0
    lax.fori_loop(0, 128, _z64, 0)

    def _z16(i, _):
        zden[i, :] = z16
        return 0
    lax.fori_loop(0, RPT, _z16, 0)

    def _zs(i, _):
        sbuf[i, :] = z16
        return 0
    lax.fori_loop(0, CE, _zs, 0)

    roff = pl.multiple_of(s * RPT, 8)
    for k in range(5):
        pltpu.sync_copy(zbuf, num_sh.at[pl.ds(roff + k * 128, 128)])
    pltpu.sync_copy(zden, den_sh.at[pl.ds(roff, RPT)])
    plsc.subcore_barrier()

    def chunk_body(ch, _):
        base = s * EPT + ch * CE
        d1 = pltpu.async_copy(src_idx.at[pl.ds(base, CE)], ii_v, sem)
        d2 = pltpu.async_copy(dst_idx.at[pl.ds(base, CE)], jj_v, sem)
        d3 = pltpu.async_copy(bias.at[pl.ds(base // 8, CE * 16 // 128)], bias_v, sem)
        d1.wait()
        d2.wait()
        d3.wait()
        g1 = pltpu.async_copy(qt.at[jj_v], qrows, sem)
        g2 = pltpu.async_copy(kt.at[ii_v], krows, sem)
        g3 = pltpu.async_copy(vt.at[ii_v], vrows, sem)
        g1.wait()
        g2.wait()
        g3.wait()

        def group_body(g, _):
            rows = g * 16 + iota16
            for h in range(HPC):
                ah = hoff + h
                flat = rows * 16 + ah
                bh = plsc.load_gather(
                    bias_v, [lax.shift_right_logical(flat, 7),
                             lax.bitwise_and(flat, 127)])
                dot = z16
                for d in range(16):
                    cc = ah * DK + d
                    ccv = jnp.broadcast_to(cc, (16,))
                    qv = plsc.load_gather(qrows, [rows, ccv])
                    kv = plsc.load_gather(krows, [rows, ccv])
                    dot = dot + qv * kv
                sh = jnp.exp(dot * 0.25 + bh)
                plsc.store_scatter(sbuf, [rows, jnp.full((16,), h, jnp.int32)], sh)
            for e in range(16):
                row = g * 16 + e
                rowv = jnp.broadcast_to(row, (16,))
                for h in range(HPC):
                    sv = plsc.load_gather(sbuf, [rowv, jnp.full((16,), h, jnp.int32)])
                    vv = vrows[row, pl.ds((hoff + h) * DK, DK)]
                    wbuf[row, pl.ds(h * DK, DK)] = vv * sv
            return 0

        lax.fori_loop(0, GP, group_body, 0)
        pltpu.sync_copy(wbuf, num_sh.at[jj_v], add=True)
        pltpu.sync_copy(sbuf, den_sh.at[jj_v], add=True)
        return 0

    lax.fori_loop(0, NCHUNK, chunk_body, 0)
    plsc.subcore_barrier()

    for k in range(5):
        pltpu.sync_copy(num_sh.at[pl.ds(roff + k * 128, 128)], zbuf)
        pltpu.sync_copy(zbuf, num_out.at[c, pl.ds(roff + k * 128, 128)])
    pltpu.sync_copy(den_sh.at[pl.ds(roff, RPT)], zden)
    pltpu.sync_copy(zden, den_out.at[c, pl.ds(roff, RPT)])


def _sc_attn(qt, kt, vt, src_idx, dst_idx, bias):
    mesh = plsc.VectorSubcoreMesh(core_axis_name="c", subcore_axis_name="s")
    return pl.kernel(
        _sc_body,
        out_type=[
            jax.ShapeDtypeStruct((NC, NPAD, 64), jnp.float32),
            jax.ShapeDtypeStruct((NC, NPAD, 16), jnp.float32),
        ],
        mesh=mesh,
        compiler_params=pltpu.CompilerParams(
            needs_layout_passes=False, use_tc_tiling_on_sc=False),
        scratch_types=[
            pltpu.VMEM((CE,), jnp.int32),
            pltpu.VMEM((CE,), jnp.int32),
            pltpu.VMEM((CE, DIM), jnp.float32),
            pltpu.VMEM((CE, DIM), jnp.float32),
            pltpu.VMEM((CE, DIM), jnp.float32),
            pltpu.VMEM((CE * 16 // 128, DIM), jnp.float32),
            pltpu.VMEM((CE, 16), jnp.float32),
            pltpu.VMEM((CE, 64), jnp.float32),
            pltpu.VMEM((128, 64), jnp.float32),
            pltpu.VMEM((RPT, 16), jnp.float32),
            pltpu.VMEM_SHARED((NPAD, 64), jnp.float32),
            pltpu.VMEM_SHARED((NPAD, 16), jnp.float32),
            pltpu.SemaphoreType.DMA,
        ],
    )(qt, kt, vt, src_idx, dst_idx, bias)


def _fin_body(num_ref, den_ref, wo_ref, bo_ref, o_ref):
    nfull = jnp.concatenate([num_ref[0], num_ref[1]], axis=1)
    dcat = jnp.concatenate([den_ref[0], den_ref[1]], axis=1)
    kk = lax.broadcasted_iota(jnp.int32, (32, DIM), 0)
    cc = lax.broadcasted_iota(jnp.int32, (32, DIM), 1)
    c16 = cc // DK
    sel = jnp.where(kk < 16, (c16 == kk).astype(jnp.float32),
                    (c16 == kk - 12).astype(jnp.float32))
    den128 = lax.dot_general(dcat, sel, (((1,), (0,)), ((), ())))
    attn = nfull / (den128 + 1e-12)
    o_ref[...] = lax.dot_general(
        attn, wo_ref[...], (((1,), (1,)), ((), ()))) + bo_ref[...]


def _finalize(num_p, den_p, wo, bo):
    bn = 1000
    return pl.pallas_call(
        _fin_body,
        grid=(N // bn,),
        in_specs=[
            pl.BlockSpec((NC, bn, 64), lambda i: (0, i, 0)),
            pl.BlockSpec((NC, bn, 16), lambda i: (0, i, 0)),
            pl.BlockSpec((DIM, DIM), lambda i: (0, 0)),
            pl.BlockSpec((1, DIM), lambda i: (0, 0)),
        ],
        out_specs=pl.BlockSpec((bn, DIM), lambda i: (i, 0)),
        out_shape=jax.ShapeDtypeStruct((N, DIM), jnp.float32),
    )(num_p, den_p, wo, bo)


def kernel(x, edge_index, edge_attr, W_Q, b_Q, W_K, b_K, W_V, b_V, W_O, b_O,
           eb_W1, eb_b1, eb_W2, eb_b2):
    ei = edge_index.astype(jnp.int32)
    qt, kt, vt = _tables(x, W_Q, b_Q[None, :], W_K, b_K[None, :], W_V, b_V[None, :])
    w2p = jnp.zeros((16, ED), jnp.float32).at[:H].set(eb_W2)
    b2p = jnp.zeros((16,), jnp.float32).at[:H].set(eb_b2)
    ebias = _edge_bias(edge_attr, eb_W1, eb_b1[None, :], w2p, b2p[None, :])
    ebias = ebias.reshape(E * 16 // 128, 128)
    num_p, den_p = _sc_attn(qt, kt, vt, ei[0], ei[1], ebias)
    return _finalize(num_p, den_p, W_O, b_O[None, :])


# speedup vs baseline: 4.4921x; 1.2063x over previous
import jax
import jax.numpy as jnp
from jax import lax
from jax.experimental import pallas as pl
from jax.experimental.pallas import tpu as pltpu
from jax.experimental.pallas import tpu_sc as plsc

N = 10000
E = 320000
DIM = 128
H = 8
DK = 16
ED = 16

NC = 2
NS = 16
NW = NC * NS
CE = 80
GP = CE // 16
NPAD = 10240
RPT = NPAD // NS


def _tables_body(x_ref, wq_ref, bq_ref, wk_ref, bk_ref, wv_ref, bv_ref,
                 q_ref, k_ref, v_ref):
    xb = x_ref[...]
    dn = (((1,), (1,)), ((), ()))
    q_ref[...] = lax.dot_general(xb, wq_ref[...], dn) + bq_ref[...]
    k_ref[...] = lax.dot_general(xb, wk_ref[...], dn) + bk_ref[...]
    v_ref[...] = lax.dot_general(xb, wv_ref[...], dn) + bv_ref[...]


def _tables(x, wq, bq, wk, bk, wv, bv):
    bn = 1000
    mspec = pl.BlockSpec((DIM, DIM), lambda i: (0, 0))
    bspec = pl.BlockSpec((1, DIM), lambda i: (0, 0))
    nspec = pl.BlockSpec((bn, DIM), lambda i: (i, 0))
    return pl.pallas_call(
        _tables_body,
        grid=(N // bn,),
        in_specs=[nspec, mspec, bspec, mspec, bspec, mspec, bspec],
        out_specs=[nspec, nspec, nspec],
        out_shape=[jax.ShapeDtypeStruct((N, DIM), jnp.float32)] * 3,
    )(x, wq, bq, wk, bk, wv, bv)


def _ebias_body(ea_ref, w1_ref, b1_ref, w2_ref, b2_ref, o_ref):
    ea = ea_ref[...]
    dn = (((1,), (1,)), ((), ()))
    h1 = lax.dot_general(ea, w1_ref[...], dn) + b1_ref[...]
    h1 = h1 * jax.nn.sigmoid(h1)
    o_ref[...] = lax.dot_general(h1, w2_ref[...], dn) + b2_ref[...]


def _edge_bias(ea, w1, b1, w2p, b2p):
    be = 20000
    return pl.pallas_call(
        _ebias_body,
        grid=(E // be,),
        in_specs=[
            pl.BlockSpec((be, ED), lambda i: (i, 0)),
            pl.BlockSpec((ED, ED), lambda i: (0, 0)),
            pl.BlockSpec((1, ED), lambda i: (0, 0)),
            pl.BlockSpec((16, ED), lambda i: (0, 0)),
            pl.BlockSpec((1, 16), lambda i: (0, 0)),
        ],
        out_specs=pl.BlockSpec((be, 16), lambda i: (i, 0)),
        out_shape=jax.ShapeDtypeStruct((E, 16), jnp.float32),
    )(ea, w1, b1, w2p, b2p)


HPC = H // NC
EPT = E // NS
NCHUNK = EPT // CE


def _sc_body(qt, kt, vt, src_idx, dst_idx, bias, num_out,
             ii_a, jj_a, qr_a, kr_a, vr_a, bv_a, wb_a,
             ii_b, jj_b, qr_b, kr_b, vr_b, bv_b, wb_b,
             num_sh, sem_a, sem_b):
    c = lax.axis_index("c")
    s = lax.axis_index("s")
    hoff = c * HPC
    z16 = jnp.zeros((16,), jnp.float32)
    iota16 = lax.iota(jnp.int32, 16)
    BR = CE * 16 // 128

    def _z80(i, _):
        for cc in range(5):
            wb_a[i, pl.ds(cc * 16, 16)] = z16
            wb_b[i, pl.ds(cc * 16, 16)] = z16
        return 0
    lax.fori_loop(0, CE, _z80, 0)

    roff = pl.multiple_of(s * RPT, 8)
    for k in range(8):
        pltpu.sync_copy(wb_a, num_sh.at[pl.ds(roff + k * CE, CE)])
    plsc.subcore_barrier()

    ebase = s * EPT

    def fire_idx(ch, ii_v, jj_v, bv_v, sem):
        base = ebase + ch * CE
        pltpu.async_copy(src_idx.at[pl.ds(base, CE)], ii_v, sem)
        pltpu.async_copy(dst_idx.at[pl.ds(base, CE)], jj_v, sem)
        pltpu.async_copy(bias.at[pl.ds(base // 8, BR)], bv_v, sem)

    def wait_idx(ch, ii_v, jj_v, bv_v, sem):
        base = ebase + ch * CE
        pltpu.make_async_copy(src_idx.at[pl.ds(base, CE)], ii_v, sem).wait()
        pltpu.make_async_copy(dst_idx.at[pl.ds(base, CE)], jj_v, sem).wait()
        pltpu.make_async_copy(bias.at[pl.ds(base // 8, BR)], bv_v, sem).wait()

    def fire_gather(ii_v, jj_v, qr, kr, vr, sem):
        pltpu.async_copy(qt.at[jj_v], qr, sem)
        pltpu.async_copy(kt.at[ii_v], kr, sem)
        pltpu.async_copy(vt.at[ii_v], vr, sem)

    def wait_gather(ii_v, jj_v, qr, kr, vr, sem):
        pltpu.make_async_copy(qt.at[jj_v], qr, sem).wait()
        pltpu.make_async_copy(kt.at[ii_v], kr, sem).wait()
        pltpu.make_async_copy(vt.at[ii_v], vr, sem).wait()

    def compute(qr, kr, vr, bv_v, wb, jj_v):
        def group_body(g, _):
            rows = g * 16 + iota16
            for h in range(HPC):
                ah = hoff + h
                flat = rows * 16 + ah
                bh = plsc.load_gather(
                    bv_v, [lax.shift_right_logical(flat, 7),
                           lax.bitwise_and(flat, 127)])
                dot = z16
                for d in range(16):
                    ccv = jnp.broadcast_to(ah * DK + d, (16,))
                    qv = plsc.load_gather(qr, [rows, ccv])
                    kv = plsc.load_gather(kr, [rows, ccv])
                    dot = dot + qv * kv
                sh = jnp.exp(dot * 0.25 + bh)
                plsc.store_scatter(wb, [rows, jnp.full((16,), 64 + h, jnp.int32)], sh)
            for e in range(16):
                row = g * 16 + e
                rowv = jnp.broadcast_to(row, (16,))
                for h in range(HPC):
                    sv = plsc.load_gather(wb, [rowv, jnp.full((16,), 64 + h, jnp.int32)])
                    vv = vr[row, pl.ds((hoff + h) * DK, DK)]
                    wb[row, pl.ds(h * DK, DK)] = vv * sv
            return 0

        lax.fori_loop(0, GP, group_body, 0)
        pltpu.sync_copy(wb, num_sh.at[jj_v], add=True)

    fire_idx(0, ii_a, jj_a, bv_a, sem_a)
    wait_idx(0, ii_a, jj_a, bv_a, sem_a)
    fire_gather(ii_a, jj_a, qr_a, kr_a, vr_a, sem_a)
    fire_idx(1, ii_b, jj_b, bv_b, sem_b)

    def pipe_body(i, _):
        e_ch = 2 * i
        wait_idx(e_ch + 1, ii_b, jj_b, bv_b, sem_b)
        fire_gather(ii_b, jj_b, qr_b, kr_b, vr_b, sem_b)
        wait_gather(ii_a, jj_a, qr_a, kr_a, vr_a, sem_a)
        compute(qr_a, kr_a, vr_a, bv_a, wb_a, jj_a)
        nxt_a = jnp.minimum(e_ch + 2, NCHUNK - 1)
        fire_idx(nxt_a, ii_a, jj_a, bv_a, sem_a)
        wait_idx(nxt_a, ii_a, jj_a, bv_a, sem_a)
        fire_gather(ii_a, jj_a, qr_a, kr_a, vr_a, sem_a)
        wait_gather(ii_b, jj_b, qr_b, kr_b, vr_b, sem_b)
        compute(qr_b, kr_b, vr_b, bv_b, wb_b, jj_b)
        nxt_b = jnp.minimum(e_ch + 3, NCHUNK - 1)
        fire_idx(nxt_b, ii_b, jj_b, bv_b, sem_b)
        return 0

    lax.fori_loop(0, NCHUNK // 2, pipe_body, 0)
    wait_idx(NCHUNK - 1, ii_b, jj_b, bv_b, sem_b)
    wait_gather(ii_a, jj_a, qr_a, kr_a, vr_a, sem_a)
    plsc.subcore_barrier()

    for k in range(8):
        pltpu.sync_copy(num_sh.at[pl.ds(roff + k * CE, CE)], wb_a)
        pltpu.sync_copy(wb_a, num_out.at[c, pl.ds(roff + k * CE, CE)])


def _sc_attn(qt, kt, vt, src_idx, dst_idx, bias):
    mesh = plsc.VectorSubcoreMesh(core_axis_name="c", subcore_axis_name="s")
    return pl.kernel(
        _sc_body,
        out_type=jax.ShapeDtypeStruct((NC, NPAD, 80), jnp.float32),
        mesh=mesh,
        compiler_params=pltpu.CompilerParams(
            needs_layout_passes=False, use_tc_tiling_on_sc=False),
        scratch_types=[
            pltpu.VMEM((CE,), jnp.int32),
            pltpu.VMEM((CE,), jnp.int32),
            pltpu.VMEM((CE, DIM), jnp.float32),
            pltpu.VMEM((CE, DIM), jnp.float32),
            pltpu.VMEM((CE, DIM), jnp.float32),
            pltpu.VMEM((CE * 16 // 128, DIM), jnp.float32),
            pltpu.VMEM((CE, 80), jnp.float32),
            pltpu.VMEM((CE,), jnp.int32),
            pltpu.VMEM((CE,), jnp.int32),
            pltpu.VMEM((CE, DIM), jnp.float32),
            pltpu.VMEM((CE, DIM), jnp.float32),
            pltpu.VMEM((CE, DIM), jnp.float32),
            pltpu.VMEM((CE * 16 // 128, DIM), jnp.float32),
            pltpu.VMEM((CE, 80), jnp.float32),
            pltpu.VMEM_SHARED((NPAD, 80), jnp.float32),
            pltpu.SemaphoreType.DMA,
            pltpu.SemaphoreType.DMA,
        ],
    )(qt, kt, vt, src_idx, dst_idx, bias)


def _fin_body(num_ref, wo_ref, bo_ref, o_ref):
    nfull = jnp.concatenate(
        [num_ref[0, :, :64], num_ref[1, :, :64]], axis=1)
    dcat = jnp.concatenate(
        [num_ref[0, :, 64:72], num_ref[1, :, 64:72]], axis=1)
    kk = lax.broadcasted_iota(jnp.int32, (16, DIM), 0)
    cc = lax.broadcasted_iota(jnp.int32, (16, DIM), 1)
    c16 = cc // DK
    sel = (kk == c16 + 4 * (c16 >= 4)).astype(jnp.float32)
    den128 = lax.dot_general(dcat, sel, (((1,), (0,)), ((), ())))
    attn = nfull / (den128 + 1e-12)
    o_ref[...] = lax.dot_general(
        attn, wo_ref[...], (((1,), (1,)), ((), ()))) + bo_ref[...]


def _finalize(num_p, wo, bo):
    bn = 1000
    return pl.pallas_call(
        _fin_body,
        grid=(N // bn,),
        in_specs=[
            pl.BlockSpec((NC, bn, 80), lambda i: (0, i, 0)),
            pl.BlockSpec((DIM, DIM), lambda i: (0, 0)),
            pl.BlockSpec((1, DIM), lambda i: (0, 0)),
        ],
        out_specs=pl.BlockSpec((bn, DIM), lambda i: (i, 0)),
        out_shape=jax.ShapeDtypeStruct((N, DIM), jnp.float32),
    )(num_p, wo, bo)


def kernel(x, edge_index, edge_attr, W_Q, b_Q, W_K, b_K, W_V, b_V, W_O, b_O,
           eb_W1, eb_b1, eb_W2, eb_b2):
    ei = edge_index.astype(jnp.int32)
    qt, kt, vt = _tables(x, W_Q, b_Q[None, :], W_K, b_K[None, :], W_V, b_V[None, :])
    w2p = jnp.zeros((16, ED), jnp.float32).at[:H].set(eb_W2)
    b2p = jnp.zeros((16,), jnp.float32).at[:H].set(eb_b2)
    ebias = _edge_bias(edge_attr, eb_W1, eb_b1[None, :], w2p, b2p[None, :])
    ebias = ebias.reshape(E * 16 // 128, 128)
    num_p = _sc_attn(qt, kt, vt, ei[0], ei[1], ebias)
    return _finalize(num_p, W_O, b_O[None, :])


# speedup vs baseline: 4.7062x; 1.0477x over previous
import jax
import jax.numpy as jnp
from jax import lax
from jax.experimental import pallas as pl
from jax.experimental.pallas import tpu as pltpu
from jax.experimental.pallas import tpu_sc as plsc

N = 10000
E = 320000
DIM = 128
H = 8
DK = 16
ED = 16

NC = 2
NS = 16
NW = NC * NS
CE = 80
GP = CE // 16
NPAD = 10240
RPT = NPAD // NS


def _tables_body(x_ref, wq_ref, bq_ref, wk_ref, bk_ref, wv_ref, bv_ref,
                 q_ref, k_ref, v_ref):
    xb = x_ref[...]
    dn = (((1,), (1,)), ((), ()))
    q_ref[...] = lax.dot_general(xb, wq_ref[...], dn) + bq_ref[...]
    k_ref[...] = lax.dot_general(xb, wk_ref[...], dn) + bk_ref[...]
    v_ref[...] = lax.dot_general(xb, wv_ref[...], dn) + bv_ref[...]


def _tables(x, wq, bq, wk, bk, wv, bv):
    bn = 1000
    mspec = pl.BlockSpec((DIM, DIM), lambda i: (0, 0))
    bspec = pl.BlockSpec((1, DIM), lambda i: (0, 0))
    nspec = pl.BlockSpec((bn, DIM), lambda i: (i, 0))
    return pl.pallas_call(
        _tables_body,
        grid=(N // bn,),
        in_specs=[nspec, mspec, bspec, mspec, bspec, mspec, bspec],
        out_specs=[nspec, nspec, nspec],
        out_shape=[jax.ShapeDtypeStruct((N, DIM), jnp.float32)] * 3,
    )(x, wq, bq, wk, bk, wv, bv)


def _ebias_body(ea_ref, w1_ref, b1_ref, w2_ref, b2_ref, o_ref):
    ea = ea_ref[...]
    dn = (((1,), (1,)), ((), ()))
    h1 = lax.dot_general(ea, w1_ref[...], dn) + b1_ref[...]
    h1 = h1 * jax.nn.sigmoid(h1)
    o_ref[...] = lax.dot_general(h1, w2_ref[...], dn) + b2_ref[...]


def _edge_bias(ea, w1, b1, w2p, b2p):
    be = 20000
    return pl.pallas_call(
        _ebias_body,
        grid=(E // be,),
        in_specs=[
            pl.BlockSpec((be, ED), lambda i: (i, 0)),
            pl.BlockSpec((ED, ED), lambda i: (0, 0)),
            pl.BlockSpec((1, ED), lambda i: (0, 0)),
            pl.BlockSpec((16, ED), lambda i: (0, 0)),
            pl.BlockSpec((1, 16), lambda i: (0, 0)),
        ],
        out_specs=pl.BlockSpec((be, 16), lambda i: (i, 0)),
        out_shape=jax.ShapeDtypeStruct((E, 16), jnp.float32),
    )(ea, w1, b1, w2p, b2p)


HPC = H // NC
EPT = E // NS
NCHUNK = EPT // CE


def _sc_body(qt, kt, vt, src_idx, dst_idx, bias, num_out,
             ii_a, jj_a, qr_a, kr_a, vr_a, bv_a, wb_a,
             ii_b, jj_b, qr_b, kr_b, vr_b, bv_b, wb_b,
             jjs_a, jjs_b, num_sh, sem_a, sem_b, sem_sa, sem_sb):
    c = lax.axis_index("c")
    s = lax.axis_index("s")
    hoff = c * HPC
    z16 = jnp.zeros((16,), jnp.float32)
    iota16 = lax.iota(jnp.int32, 16)
    BR = CE * 16 // 128

    def _z80(i, _):
        for cc in range(5):
            wb_a[i, pl.ds(cc * 16, 16)] = z16
            wb_b[i, pl.ds(cc * 16, 16)] = z16
        return 0
    lax.fori_loop(0, CE, _z80, 0)

    roff = pl.multiple_of(s * RPT, 8)
    for k in range(8):
        pltpu.sync_copy(wb_a, num_sh.at[pl.ds(roff + k * CE, CE)])
    plsc.subcore_barrier()

    ebase = s * EPT

    def fire_idx(ch, ii_v, jj_v, bv_v, sem):
        base = ebase + ch * CE
        pltpu.async_copy(src_idx.at[pl.ds(base, CE)], ii_v, sem)
        pltpu.async_copy(dst_idx.at[pl.ds(base, CE)], jj_v, sem)
        pltpu.async_copy(bias.at[pl.ds(base // 8, BR)], bv_v, sem)

    def wait_idx(ch, ii_v, jj_v, bv_v, sem):
        base = ebase + ch * CE
        pltpu.make_async_copy(src_idx.at[pl.ds(base, CE)], ii_v, sem).wait()
        pltpu.make_async_copy(dst_idx.at[pl.ds(base, CE)], jj_v, sem).wait()
        pltpu.make_async_copy(bias.at[pl.ds(base // 8, BR)], bv_v, sem).wait()

    def fire_gather(ii_v, jj_v, qr, kr, vr, sem):
        pltpu.async_copy(qt.at[jj_v], qr, sem)
        pltpu.async_copy(kt.at[ii_v], kr, sem)
        pltpu.async_copy(vt.at[ii_v], vr, sem)

    def wait_gather(ii_v, jj_v, qr, kr, vr, sem):
        pltpu.make_async_copy(qt.at[jj_v], qr, sem).wait()
        pltpu.make_async_copy(kt.at[ii_v], kr, sem).wait()
        pltpu.make_async_copy(vt.at[ii_v], vr, sem).wait()

    def compute(qr, kr, vr, bv_v, wb):
        def group_body(g, _):
            rows = g * 16 + iota16
            for h in range(HPC):
                ah = hoff + h
                flat = rows * 16 + ah
                bh = plsc.load_gather(
                    bv_v, [lax.shift_right_logical(flat, 7),
                           lax.bitwise_and(flat, 127)])
                dots = [z16, z16, z16, z16]
                for d in range(16):
                    ccv = jnp.broadcast_to(ah * DK + d, (16,))
                    qv = plsc.load_gather(qr, [rows, ccv])
                    kv = plsc.load_gather(kr, [rows, ccv])
                    dots[d % 4] = dots[d % 4] + qv * kv
                dot = (dots[0] + dots[1]) + (dots[2] + dots[3])
                sh = jnp.exp(dot * 0.25 + bh)
                plsc.store_scatter(wb, [rows, jnp.full((16,), 64 + h, jnp.int32)], sh)
            for e in range(16):
                row = g * 16 + e
                rowv = jnp.broadcast_to(row, (16,))
                for h in range(HPC):
                    sv = plsc.load_gather(wb, [rowv, jnp.full((16,), 64 + h, jnp.int32)])
                    vv = vr[row, pl.ds((hoff + h) * DK, DK)]
                    wb[row, pl.ds(h * DK, DK)] = vv * sv
            return 0

        lax.fori_loop(0, GP, group_body, 0)

    def snap_jj(jj_v, jjs_v):
        for k in range(GP):
            jjs_v[pl.ds(k * 16, 16)] = jj_v[pl.ds(k * 16, 16)]

    def fire_scatter(wb, jjs_v, sem):
        pltpu.async_copy(wb, num_sh.at[jjs_v], sem, add=True)

    def wait_scatter(wb, jjs_v, sem):
        pltpu.make_async_copy(wb, num_sh.at[jjs_v], sem).wait()

    fire_idx(0, ii_a, jj_a, bv_a, sem_a)
    wait_idx(0, ii_a, jj_a, bv_a, sem_a)
    fire_gather(ii_a, jj_a, qr_a, kr_a, vr_a, sem_a)
    fire_idx(1, ii_b, jj_b, bv_b, sem_b)

    def pipe_body(i, _):
        e_ch = 2 * i
        wait_idx(e_ch + 1, ii_b, jj_b, bv_b, sem_b)
        fire_gather(ii_b, jj_b, qr_b, kr_b, vr_b, sem_b)
        wait_gather(ii_a, jj_a, qr_a, kr_a, vr_a, sem_a)

        @pl.when(i > 0)
        def _():
            wait_scatter(wb_a, jjs_a, sem_sa)
        compute(qr_a, kr_a, vr_a, bv_a, wb_a)
        snap_jj(jj_a, jjs_a)
        fire_scatter(wb_a, jjs_a, sem_sa)
        nxt_a = jnp.minimum(e_ch + 2, NCHUNK - 1)
        fire_idx(nxt_a, ii_a, jj_a, bv_a, sem_a)
        wait_idx(nxt_a, ii_a, jj_a, bv_a, sem_a)
        fire_gather(ii_a, jj_a, qr_a, kr_a, vr_a, sem_a)
        wait_gather(ii_b, jj_b, qr_b, kr_b, vr_b, sem_b)

        @pl.when(i > 0)
        def _():
            wait_scatter(wb_b, jjs_b, sem_sb)
        compute(qr_b, kr_b, vr_b, bv_b, wb_b)
        snap_jj(jj_b, jjs_b)
        fire_scatter(wb_b, jjs_b, sem_sb)
        nxt_b = jnp.minimum(e_ch + 3, NCHUNK - 1)
        fire_idx(nxt_b, ii_b, jj_b, bv_b, sem_b)
        return 0

    lax.fori_loop(0, NCHUNK // 2, pipe_body, 0)
    wait_idx(NCHUNK - 1, ii_b, jj_b, bv_b, sem_b)
    wait_gather(ii_a, jj_a, qr_a, kr_a, vr_a, sem_a)
    wait_scatter(wb_a, jjs_a, sem_sa)
    wait_scatter(wb_b, jjs_b, sem_sb)
    plsc.subcore_barrier()

    for k in range(8):
        pltpu.sync_copy(num_sh.at[pl.ds(roff + k * CE, CE)], wb_a)
        pltpu.sync_copy(wb_a, num_out.at[c, pl.ds(roff + k * CE, CE)])


def _sc_attn(qt, kt, vt, src_idx, dst_idx, bias):
    mesh = plsc.VectorSubcoreMesh(core_axis_name="c", subcore_axis_name="s")
    return pl.kernel(
        _sc_body,
        out_type=jax.ShapeDtypeStruct((NC, NPAD, 80), jnp.float32),
        mesh=mesh,
        compiler_params=pltpu.CompilerParams(
            needs_layout_passes=False, use_tc_tiling_on_sc=False),
        scratch_types=[
            pltpu.VMEM((CE,), jnp.int32),
            pltpu.VMEM((CE,), jnp.int32),
            pltpu.VMEM((CE, DIM), jnp.float32),
            pltpu.VMEM((CE, DIM), jnp.float32),
            pltpu.VMEM((CE, DIM), jnp.float32),
            pltpu.VMEM((CE * 16 // 128, DIM), jnp.float32),
            pltpu.VMEM((CE, 80), jnp.float32),
            pltpu.VMEM((CE,), jnp.int32),
            pltpu.VMEM((CE,), jnp.int32),
            pltpu.VMEM((CE, DIM), jnp.float32),
            pltpu.VMEM((CE, DIM), jnp.float32),
            pltpu.VMEM((CE, DIM), jnp.float32),
            pltpu.VMEM((CE * 16 // 128, DIM), jnp.float32),
            pltpu.VMEM((CE, 80), jnp.float32),
            pltpu.VMEM((CE,), jnp.int32),
            pltpu.VMEM((CE,), jnp.int32),
            pltpu.VMEM_SHARED((NPAD, 80), jnp.float32),
            pltpu.SemaphoreType.DMA,
            pltpu.SemaphoreType.DMA,
            pltpu.SemaphoreType.DMA,
            pltpu.SemaphoreType.DMA,
        ],
    )(qt, kt, vt, src_idx, dst_idx, bias)


def _fin_body(num_ref, wo_ref, bo_ref, o_ref):
    nfull = jnp.concatenate(
        [num_ref[0, :, :64], num_ref[1, :, :64]], axis=1)
    dcat = jnp.concatenate(
        [num_ref[0, :, 64:72], num_ref[1, :, 64:72]], axis=1)
    kk = lax.broadcasted_iota(jnp.int32, (16, DIM), 0)
    cc = lax.broadcasted_iota(jnp.int32, (16, DIM), 1)
    c16 = cc // DK
    sel = (kk == c16 + 4 * (c16 >= 4)).astype(jnp.float32)
    den128 = lax.dot_general(dcat, sel, (((1,), (0,)), ((), ())))
    attn = nfull / (den128 + 1e-12)
    o_ref[...] = lax.dot_general(
        attn, wo_ref[...], (((1,), (1,)), ((), ()))) + bo_ref[...]


def _finalize(num_p, wo, bo):
    bn = 1000
    return pl.pallas_call(
        _fin_body,
        grid=(N // bn,),
        in_specs=[
            pl.BlockSpec((NC, bn, 80), lambda i: (0, i, 0)),
            pl.BlockSpec((DIM, DIM), lambda i: (0, 0)),
            pl.BlockSpec((1, DIM), lambda i: (0, 0)),
        ],
        out_specs=pl.BlockSpec((bn, DIM), lambda i: (i, 0)),
        out_shape=jax.ShapeDtypeStruct((N, DIM), jnp.float32),
    )(num_p, wo, bo)


def kernel(x, edge_index, edge_attr, W_Q, b_Q, W_K, b_K, W_V, b_V, W_O, b_O,
           eb_W1, eb_b1, eb_W2, eb_b2):
    ei = edge_index.astype(jnp.int32)
    qt, kt, vt = _tables(x, W_Q, b_Q[None, :], W_K, b_K[None, :], W_V, b_V[None, :])
    w2p = jnp.zeros((16, ED), jnp.float32).at[:H].set(eb_W2)
    b2p = jnp.zeros((16,), jnp.float32).at[:H].set(eb_b2)
    ebias = _edge_bias(edge_attr, eb_W1, eb_b1[None, :], w2p, b2p[None, :])
    ebias = ebias.reshape(E * 16 // 128, 128)
    num_p = _sc_attn(qt, kt, vt, ei[0], ei[1], ebias)
    return _finalize(num_p, W_O, b_O[None, :])


# speedup vs baseline: 5.3711x; 1.1413x over previous
import jax
import jax.numpy as jnp
from jax import lax
from jax.experimental import pallas as pl
from jax.experimental.pallas import tpu as pltpu
from jax.experimental.pallas import tpu_sc as plsc

N = 10000
E = 320000
DIM = 128
H = 8
DK = 16
ED = 16

NC = 2
NS = 16
NW = NC * NS
CE = 80
GP = CE // 16
NPAD = 10240
RPT = NPAD // NS


def _tables_body(x_ref, wq_ref, bq_ref, wk_ref, bk_ref, wv_ref, bv_ref,
                 q_ref, k_ref, v_ref):
    xb = x_ref[...]
    dn = (((1,), (1,)), ((), ()))
    q_ref[...] = lax.dot_general(xb, wq_ref[...], dn) + bq_ref[...]
    k_ref[...] = lax.dot_general(xb, wk_ref[...], dn) + bk_ref[...]
    v_ref[...] = lax.dot_general(xb, wv_ref[...], dn) + bv_ref[...]


def _tables(x, wq, bq, wk, bk, wv, bv):
    bn = 1000
    mspec = pl.BlockSpec((DIM, DIM), lambda i: (0, 0))
    bspec = pl.BlockSpec((1, DIM), lambda i: (0, 0))
    nspec = pl.BlockSpec((bn, DIM), lambda i: (i, 0))
    return pl.pallas_call(
        _tables_body,
        grid=(N // bn,),
        in_specs=[nspec, mspec, bspec, mspec, bspec, mspec, bspec],
        out_specs=[nspec, nspec, nspec],
        out_shape=[jax.ShapeDtypeStruct((N, DIM), jnp.float32)] * 3,
    )(x, wq, bq, wk, bk, wv, bv)


def _ebias_body(ea_ref, w1_ref, b1_ref, w2_ref, b2_ref, o_ref):
    ea = ea_ref[...]
    dn = (((1,), (1,)), ((), ()))
    h1 = lax.dot_general(ea, w1_ref[...], dn) + b1_ref[...]
    h1 = h1 * jax.nn.sigmoid(h1)
    o_ref[...] = lax.dot_general(h1, w2_ref[...], dn) + b2_ref[...]


def _edge_bias(ea, w1, b1, w2p, b2p):
    be = 20000
    return pl.pallas_call(
        _ebias_body,
        grid=(E // be,),
        in_specs=[
            pl.BlockSpec((be, ED), lambda i: (i, 0)),
            pl.BlockSpec((ED, ED), lambda i: (0, 0)),
            pl.BlockSpec((1, ED), lambda i: (0, 0)),
            pl.BlockSpec((16, ED), lambda i: (0, 0)),
            pl.BlockSpec((1, 16), lambda i: (0, 0)),
        ],
        out_specs=pl.BlockSpec((be, 16), lambda i: (i, 0)),
        out_shape=jax.ShapeDtypeStruct((E, 16), jnp.float32),
    )(ea, w1, b1, w2p, b2p)


HPC = H // NC
EPT = E // NS
NCHUNK = EPT // CE


def _sc_body(qt, kt, vt, src_idx, dst_idx, bias, num_out,
             ii_a, jj_a, qr_a, kr_a, vr_a, bv_a, wb_a,
             ii_b, jj_b, qr_b, kr_b, vr_b, bv_b, wb_b,
             jjs_a, jjs_b, num_sh, sem_a, sem_b, sem_sa, sem_sb):
    c = lax.axis_index("c")
    s = lax.axis_index("s")
    hoff = c * HPC
    z16 = jnp.zeros((16,), jnp.float32)
    iota16 = lax.iota(jnp.int32, 16)
    BR = CE * 16 // 128

    def _z80(i, _):
        for cc in range(5):
            wb_a[i, pl.ds(cc * 16, 16)] = z16
            wb_b[i, pl.ds(cc * 16, 16)] = z16
        return 0
    lax.fori_loop(0, CE, _z80, 0)

    roff = pl.multiple_of(s * RPT, 8)
    for k in range(8):
        pltpu.sync_copy(wb_a, num_sh.at[pl.ds(roff + k * CE, CE)])
    plsc.subcore_barrier()

    ebase = s * EPT

    def fire_idx(ch, ii_v, jj_v, bv_v, sem):
        base = ebase + ch * CE
        pltpu.async_copy(src_idx.at[pl.ds(base, CE)], ii_v, sem)
        pltpu.async_copy(dst_idx.at[pl.ds(base, CE)], jj_v, sem)
        pltpu.async_copy(bias.at[pl.ds(base // 8, BR)], bv_v, sem)

    def wait_idx(ch, ii_v, jj_v, bv_v, sem):
        base = ebase + ch * CE
        pltpu.make_async_copy(src_idx.at[pl.ds(base, CE)], ii_v, sem).wait()
        pltpu.make_async_copy(dst_idx.at[pl.ds(base, CE)], jj_v, sem).wait()
        pltpu.make_async_copy(bias.at[pl.ds(base // 8, BR)], bv_v, sem).wait()

    def fire_gather(ii_v, jj_v, qr, kr, vr, sem):
        pltpu.async_copy(qt.at[jj_v], qr, sem)
        pltpu.async_copy(kt.at[ii_v], kr, sem)
        pltpu.async_copy(vt.at[ii_v], vr, sem)

    def wait_gather(ii_v, jj_v, qr, kr, vr, sem):
        pltpu.make_async_copy(qt.at[jj_v], qr, sem).wait()
        pltpu.make_async_copy(kt.at[ii_v], kr, sem).wait()
        pltpu.make_async_copy(vt.at[ii_v], vr, sem).wait()

    lane15 = iota16 == 15

    def compute(qr, kr, vr, bv_v, wb):
        def group_body(g, _):
            for e in range(16):
                row = g * 16 + e
                rowv = jnp.broadcast_to(row, (16,))
                for h in range(HPC):
                    off = (hoff + h) * DK
                    qv = qr[row, pl.ds(off, DK)]
                    kv = kr[row, pl.ds(off, DK)]
                    cs = plsc.cumsum(qv * kv)
                    plsc.store_scatter(
                        wb, [rowv, jnp.full((16,), 64 + h, jnp.int32)], cs,
                        mask=lane15)
            rows = g * 16 + iota16
            for h in range(HPC):
                ah = hoff + h
                flat = rows * 16 + ah
                bh = plsc.load_gather(
                    bv_v, [lax.shift_right_logical(flat, 7),
                           lax.bitwise_and(flat, 127)])
                hcol = jnp.full((16,), 64 + h, jnp.int32)
                dv = plsc.load_gather(wb, [rows, hcol])
                sh = jnp.exp(dv * 0.25 + bh)
                plsc.store_scatter(wb, [rows, hcol], sh)
            for e in range(16):
                row = g * 16 + e
                rowv = jnp.broadcast_to(row, (16,))
                for h in range(HPC):
                    sv = plsc.load_gather(wb, [rowv, jnp.full((16,), 64 + h, jnp.int32)])
                    vv = vr[row, pl.ds((hoff + h) * DK, DK)]
                    wb[row, pl.ds(h * DK, DK)] = vv * sv
            return 0

        lax.fori_loop(0, GP, group_body, 0)

    def snap_jj(jj_v, jjs_v):
        for k in range(GP):
            jjs_v[pl.ds(k * 16, 16)] = jj_v[pl.ds(k * 16, 16)]

    def fire_scatter(wb, jjs_v, sem):
        pltpu.async_copy(wb, num_sh.at[jjs_v], sem, add=True)

    def wait_scatter(wb, jjs_v, sem):
        pltpu.make_async_copy(wb, num_sh.at[jjs_v], sem).wait()

    fire_idx(0, ii_a, jj_a, bv_a, sem_a)
    wait_idx(0, ii_a, jj_a, bv_a, sem_a)
    fire_gather(ii_a, jj_a, qr_a, kr_a, vr_a, sem_a)
    fire_idx(1, ii_b, jj_b, bv_b, sem_b)

    def pipe_body(i, _):
        e_ch = 2 * i
        wait_idx(e_ch + 1, ii_b, jj_b, bv_b, sem_b)
        fire_gather(ii_b, jj_b, qr_b, kr_b, vr_b, sem_b)
        wait_gather(ii_a, jj_a, qr_a, kr_a, vr_a, sem_a)

        @pl.when(i > 0)
        def _():
            wait_scatter(wb_a, jjs_a, sem_sa)
        compute(qr_a, kr_a, vr_a, bv_a, wb_a)
        snap_jj(jj_a, jjs_a)
        fire_scatter(wb_a, jjs_a, sem_sa)
        nxt_a = jnp.minimum(e_ch + 2, NCHUNK - 1)
        fire_idx(nxt_a, ii_a, jj_a, bv_a, sem_a)
        wait_idx(nxt_a, ii_a, jj_a, bv_a, sem_a)
        fire_gather(ii_a, jj_a, qr_a, kr_a, vr_a, sem_a)
        wait_gather(ii_b, jj_b, qr_b, kr_b, vr_b, sem_b)

        @pl.when(i > 0)
        def _():
            wait_scatter(wb_b, jjs_b, sem_sb)
        compute(qr_b, kr_b, vr_b, bv_b, wb_b)
        snap_jj(jj_b, jjs_b)
        fire_scatter(wb_b, jjs_b, sem_sb)
        nxt_b = jnp.minimum(e_ch + 3, NCHUNK - 1)
        fire_idx(nxt_b, ii_b, jj_b, bv_b, sem_b)
        return 0

    lax.fori_loop(0, NCHUNK // 2, pipe_body, 0)
    wait_idx(NCHUNK - 1, ii_b, jj_b, bv_b, sem_b)
    wait_gather(ii_a, jj_a, qr_a, kr_a, vr_a, sem_a)
    wait_scatter(wb_a, jjs_a, sem_sa)
    wait_scatter(wb_b, jjs_b, sem_sb)
    plsc.subcore_barrier()

    for k in range(8):
        pltpu.sync_copy(num_sh.at[pl.ds(roff + k * CE, CE)], wb_a)
        pltpu.sync_copy(wb_a, num_out.at[c, pl.ds(roff + k * CE, CE)])


def _sc_attn(qt, kt, vt, src_idx, dst_idx, bias):
    mesh = plsc.VectorSubcoreMesh(core_axis_name="c", subcore_axis_name="s")
    return pl.kernel(
        _sc_body,
        out_type=jax.ShapeDtypeStruct((NC, NPAD, 80), jnp.float32),
        mesh=mesh,
        compiler_params=pltpu.CompilerParams(
            needs_layout_passes=False, use_tc_tiling_on_sc=False),
        scratch_types=[
            pltpu.VMEM((CE,), jnp.int32),
            pltpu.VMEM((CE,), jnp.int32),
            pltpu.VMEM((CE, DIM), jnp.float32),
            pltpu.VMEM((CE, DIM), jnp.float32),
            pltpu.VMEM((CE, DIM), jnp.float32),
            pltpu.VMEM((CE * 16 // 128, DIM), jnp.float32),
            pltpu.VMEM((CE, 80), jnp.float32),
            pltpu.VMEM((CE,), jnp.int32),
            pltpu.VMEM((CE,), jnp.int32),
            pltpu.VMEM((CE, DIM), jnp.float32),
            pltpu.VMEM((CE, DIM), jnp.float32),
            pltpu.VMEM((CE, DIM), jnp.float32),
            pltpu.VMEM((CE * 16 // 128, DIM), jnp.float32),
            pltpu.VMEM((CE, 80), jnp.float32),
            pltpu.VMEM((CE,), jnp.int32),
            pltpu.VMEM((CE,), jnp.int32),
            pltpu.VMEM_SHARED((NPAD, 80), jnp.float32),
            pltpu.SemaphoreType.DMA,
            pltpu.SemaphoreType.DMA,
            pltpu.SemaphoreType.DMA,
            pltpu.SemaphoreType.DMA,
        ],
    )(qt, kt, vt, src_idx, dst_idx, bias)


def _fin_body(num_ref, wo_ref, bo_ref, o_ref):
    nfull = jnp.concatenate(
        [num_ref[0, :, :64], num_ref[1, :, :64]], axis=1)
    dcat = jnp.concatenate(
        [num_ref[0, :, 64:72], num_ref[1, :, 64:72]], axis=1)
    kk = lax.broadcasted_iota(jnp.int32, (16, DIM), 0)
    cc = lax.broadcasted_iota(jnp.int32, (16, DIM), 1)
    c16 = cc // DK
    sel = (kk == c16 + 4 * (c16 >= 4)).astype(jnp.float32)
    den128 = lax.dot_general(dcat, sel, (((1,), (0,)), ((), ())))
    attn = nfull / (den128 + 1e-12)
    o_ref[...] = lax.dot_general(
        attn, wo_ref[...], (((1,), (1,)), ((), ()))) + bo_ref[...]


def _finalize(num_p, wo, bo):
    bn = 1000
    return pl.pallas_call(
        _fin_body,
        grid=(N // bn,),
        in_specs=[
            pl.BlockSpec((NC, bn, 80), lambda i: (0, i, 0)),
            pl.BlockSpec((DIM, DIM), lambda i: (0, 0)),
            pl.BlockSpec((1, DIM), lambda i: (0, 0)),
        ],
        out_specs=pl.BlockSpec((bn, DIM), lambda i: (i, 0)),
        out_shape=jax.ShapeDtypeStruct((N, DIM), jnp.float32),
    )(num_p, wo, bo)


def kernel(x, edge_index, edge_attr, W_Q, b_Q, W_K, b_K, W_V, b_V, W_O, b_O,
           eb_W1, eb_b1, eb_W2, eb_b2):
    ei = edge_index.astype(jnp.int32)
    qt, kt, vt = _tables(x, W_Q, b_Q[None, :], W_K, b_K[None, :], W_V, b_V[None, :])
    w2p = jnp.zeros((16, ED), jnp.float32).at[:H].set(eb_W2)
    b2p = jnp.zeros((16,), jnp.float32).at[:H].set(eb_b2)
    ebias = _edge_bias(edge_attr, eb_W1, eb_b1[None, :], w2p, b2p[None, :])
    ebias = ebias.reshape(E * 16 // 128, 128)
    num_p = _sc_attn(qt, kt, vt, ei[0], ei[1], ebias)
    return _finalize(num_p, W_O, b_O[None, :])


# speedup vs baseline: 5.3876x; 1.0031x over previous
import jax
import jax.numpy as jnp
from jax import lax
from jax.experimental import pallas as pl
from jax.experimental.pallas import tpu as pltpu
from jax.experimental.pallas import tpu_sc as plsc

N = 10000
E = 320000
DIM = 128
H = 8
DK = 16
ED = 16

NC = 2
NS = 16
NW = NC * NS
CE = 80
GP = CE // 16
NPAD = 10240
RPT = NPAD // NS


def _tables_body(x_ref, wq_ref, bq_ref, wk_ref, bk_ref, wv_ref, bv_ref,
                 q_ref, k_ref, v_ref):
    xb = x_ref[...]
    dn = (((1,), (1,)), ((), ()))
    q_ref[...] = (lax.dot_general(xb, wq_ref[...], dn) + bq_ref[...]) * 0.25
    k_ref[...] = lax.dot_general(xb, wk_ref[...], dn) + bk_ref[...]
    v_ref[...] = lax.dot_general(xb, wv_ref[...], dn) + bv_ref[...]


def _tables(x, wq, bq, wk, bk, wv, bv):
    bn = 1000
    mspec = pl.BlockSpec((DIM, DIM), lambda i: (0, 0))
    bspec = pl.BlockSpec((1, DIM), lambda i: (0, 0))
    nspec = pl.BlockSpec((bn, DIM), lambda i: (i, 0))
    return pl.pallas_call(
        _tables_body,
        grid=(N // bn,),
        in_specs=[nspec, mspec, bspec, mspec, bspec, mspec, bspec],
        out_specs=[nspec, nspec, nspec],
        out_shape=[jax.ShapeDtypeStruct((N, DIM), jnp.float32)] * 3,
    )(x, wq, bq, wk, bk, wv, bv)


def _ebias_body(ea_ref, w1_ref, b1_ref, w2_ref, b2_ref, o_ref):
    ea = ea_ref[...]
    dn = (((1,), (1,)), ((), ()))
    h1 = lax.dot_general(ea, w1_ref[...], dn) + b1_ref[...]
    h1 = h1 * jax.nn.sigmoid(h1)
    o_ref[...] = lax.dot_general(h1, w2_ref[...], dn) + b2_ref[...]


def _edge_bias(ea, w1, b1, w2p, b2p):
    be = 20000
    return pl.pallas_call(
        _ebias_body,
        grid=(E // be,),
        in_specs=[
            pl.BlockSpec((be, ED), lambda i: (i, 0)),
            pl.BlockSpec((ED, ED), lambda i: (0, 0)),
            pl.BlockSpec((1, ED), lambda i: (0, 0)),
            pl.BlockSpec((16, ED), lambda i: (0, 0)),
            pl.BlockSpec((1, 16), lambda i: (0, 0)),
        ],
        out_specs=pl.BlockSpec((be, 16), lambda i: (i, 0)),
        out_shape=jax.ShapeDtypeStruct((E, 16), jnp.float32),
    )(ea, w1, b1, w2p, b2p)


HPC = H // NC
EPT = E // NS
NCHUNK = EPT // CE


def _sc_body(qt, kt, vt, src_idx, dst_idx, bias, num_out,
             ii_a, jj_a, qr_a, kr_a, vr_a, bv_a, wb_a,
             ii_b, jj_b, qr_b, kr_b, vr_b, bv_b, wb_b,
             jjs_a, jjs_b, num_sh, sem_a, sem_b, sem_sa, sem_sb):
    c = lax.axis_index("c")
    s = lax.axis_index("s")
    hoff = c * HPC
    z16 = jnp.zeros((16,), jnp.float32)
    iota16 = lax.iota(jnp.int32, 16)
    BR = CE * 16 // 128

    def _z80(i, _):
        for cc in range(5):
            wb_a[i, pl.ds(cc * 16, 16)] = z16
            wb_b[i, pl.ds(cc * 16, 16)] = z16
        return 0
    lax.fori_loop(0, CE, _z80, 0)

    roff = pl.multiple_of(s * RPT, 8)
    for k in range(8):
        pltpu.sync_copy(wb_a, num_sh.at[pl.ds(roff + k * CE, CE)])
    plsc.subcore_barrier()

    ebase = s * EPT

    def fire_idx(ch, ii_v, jj_v, bv_v, sem):
        base = ebase + ch * CE
        pltpu.async_copy(src_idx.at[pl.ds(base, CE)], ii_v, sem)
        pltpu.async_copy(dst_idx.at[pl.ds(base, CE)], jj_v, sem)
        pltpu.async_copy(bias.at[pl.ds(base // 8, BR)], bv_v, sem)

    def wait_idx(ch, ii_v, jj_v, bv_v, sem):
        base = ebase + ch * CE
        pltpu.make_async_copy(src_idx.at[pl.ds(base, CE)], ii_v, sem).wait()
        pltpu.make_async_copy(dst_idx.at[pl.ds(base, CE)], jj_v, sem).wait()
        pltpu.make_async_copy(bias.at[pl.ds(base // 8, BR)], bv_v, sem).wait()

    def fire_gather(ii_v, jj_v, qr, kr, vr, sem):
        pltpu.async_copy(qt.at[jj_v], qr, sem)
        pltpu.async_copy(kt.at[ii_v], kr, sem)
        pltpu.async_copy(vt.at[ii_v], vr, sem)

    def wait_gather(ii_v, jj_v, qr, kr, vr, sem):
        pltpu.make_async_copy(qt.at[jj_v], qr, sem).wait()
        pltpu.make_async_copy(kt.at[ii_v], kr, sem).wait()
        pltpu.make_async_copy(vt.at[ii_v], vr, sem).wait()

    lane15 = iota16 == 15

    def compute(qr, kr, vr, bv_v, wb):
        def group_body(g, _):
            for h in range(HPC):
                off = (hoff + h) * DK
                hcolv = jnp.full((16,), 64 + h, jnp.int32)
                for e in range(16):
                    row = g * 16 + e
                    rowv = jnp.broadcast_to(row, (16,))
                    qv = qr[row, pl.ds(off, DK)]
                    kv = kr[row, pl.ds(off, DK)]
                    cs = plsc.cumsum(qv * kv)
                    plsc.store_scatter(wb, [rowv, hcolv], cs, mask=lane15)
            rows = g * 16 + iota16
            for h in range(HPC):
                ah = hoff + h
                flat = rows * 16 + ah
                bh = plsc.load_gather(
                    bv_v, [lax.shift_right_logical(flat, 7),
                           lax.bitwise_and(flat, 127)])
                hcol = jnp.full((16,), 64 + h, jnp.int32)
                dv = plsc.load_gather(wb, [rows, hcol])
                sh = jnp.exp(dv + bh)
                plsc.store_scatter(wb, [rows, hcol], sh)
            for e in range(16):
                row = g * 16 + e
                rowv = jnp.broadcast_to(row, (16,))
                for h in range(HPC):
                    sv = plsc.load_gather(wb, [rowv, jnp.full((16,), 64 + h, jnp.int32)])
                    vv = vr[row, pl.ds((hoff + h) * DK, DK)]
                    wb[row, pl.ds(h * DK, DK)] = vv * sv
            return 0

        lax.fori_loop(0, GP, group_body, 0)

    def snap_jj(jj_v, jjs_v):
        for k in range(GP):
            jjs_v[pl.ds(k * 16, 16)] = jj_v[pl.ds(k * 16, 16)]

    def fire_scatter(wb, jjs_v, sem):
        pltpu.async_copy(wb, num_sh.at[jjs_v], sem, add=True)

    def wait_scatter(wb, jjs_v, sem):
        pltpu.make_async_copy(wb, num_sh.at[jjs_v], sem).wait()

    fire_idx(0, ii_a, jj_a, bv_a, sem_a)
    wait_idx(0, ii_a, jj_a, bv_a, sem_a)
    fire_gather(ii_a, jj_a, qr_a, kr_a, vr_a, sem_a)
    fire_idx(1, ii_b, jj_b, bv_b, sem_b)

    def pipe_body(i, _):
        e_ch = 2 * i
        wait_idx(e_ch + 1, ii_b, jj_b, bv_b, sem_b)
        fire_gather(ii_b, jj_b, qr_b, kr_b, vr_b, sem_b)
        wait_gather(ii_a, jj_a, qr_a, kr_a, vr_a, sem_a)

        @pl.when(i > 0)
        def _():
            wait_scatter(wb_a, jjs_a, sem_sa)
        compute(qr_a, kr_a, vr_a, bv_a, wb_a)
        snap_jj(jj_a, jjs_a)
        fire_scatter(wb_a, jjs_a, sem_sa)
        nxt_a = jnp.minimum(e_ch + 2, NCHUNK - 1)
        fire_idx(nxt_a, ii_a, jj_a, bv_a, sem_a)
        wait_idx(nxt_a, ii_a, jj_a, bv_a, sem_a)
        fire_gather(ii_a, jj_a, qr_a, kr_a, vr_a, sem_a)
        wait_gather(ii_b, jj_b, qr_b, kr_b, vr_b, sem_b)

        @pl.when(i > 0)
        def _():
            wait_scatter(wb_b, jjs_b, sem_sb)
        compute(qr_b, kr_b, vr_b, bv_b, wb_b)
        snap_jj(jj_b, jjs_b)
        fire_scatter(wb_b, jjs_b, sem_sb)
        nxt_b = jnp.minimum(e_ch + 3, NCHUNK - 1)
        fire_idx(nxt_b, ii_b, jj_b, bv_b, sem_b)
        return 0

    lax.fori_loop(0, NCHUNK // 2, pipe_body, 0)
    wait_idx(NCHUNK - 1, ii_b, jj_b, bv_b, sem_b)
    wait_gather(ii_a, jj_a, qr_a, kr_a, vr_a, sem_a)
    wait_scatter(wb_a, jjs_a, sem_sa)
    wait_scatter(wb_b, jjs_b, sem_sb)
    plsc.subcore_barrier()

    for k in range(8):
        pltpu.sync_copy(num_sh.at[pl.ds(roff + k * CE, CE)], wb_a)
        pltpu.sync_copy(wb_a, num_out.at[c, pl.ds(roff + k * CE, CE)])


def _sc_attn(qt, kt, vt, src_idx, dst_idx, bias):
    mesh = plsc.VectorSubcoreMesh(core_axis_name="c", subcore_axis_name="s")
    return pl.kernel(
        _sc_body,
        out_type=jax.ShapeDtypeStruct((NC, NPAD, 80), jnp.float32),
        mesh=mesh,
        compiler_params=pltpu.CompilerParams(
            needs_layout_passes=False, use_tc_tiling_on_sc=False),
        scratch_types=[
            pltpu.VMEM((CE,), jnp.int32),
            pltpu.VMEM((CE,), jnp.int32),
            pltpu.VMEM((CE, DIM), jnp.float32),
            pltpu.VMEM((CE, DIM), jnp.float32),
            pltpu.VMEM((CE, DIM), jnp.float32),
            pltpu.VMEM((CE * 16 // 128, DIM), jnp.float32),
            pltpu.VMEM((CE, 80), jnp.float32),
            pltpu.VMEM((CE,), jnp.int32),
            pltpu.VMEM((CE,), jnp.int32),
            pltpu.VMEM((CE, DIM), jnp.float32),
            pltpu.VMEM((CE, DIM), jnp.float32),
            pltpu.VMEM((CE, DIM), jnp.float32),
            pltpu.VMEM((CE * 16 // 128, DIM), jnp.float32),
            pltpu.VMEM((CE, 80), jnp.float32),
            pltpu.VMEM((CE,), jnp.int32),
            pltpu.VMEM((CE,), jnp.int32),
            pltpu.VMEM_SHARED((NPAD, 80), jnp.float32),
            pltpu.SemaphoreType.DMA,
            pltpu.SemaphoreType.DMA,
            pltpu.SemaphoreType.DMA,
            pltpu.SemaphoreType.DMA,
        ],
    )(qt, kt, vt, src_idx, dst_idx, bias)


def _fin_body(num_ref, wo_ref, bo_ref, o_ref):
    nfull = jnp.concatenate(
        [num_ref[0, :, :64], num_ref[1, :, :64]], axis=1)
    dcat = jnp.concatenate(
        [num_ref[0, :, 64:72], num_ref[1, :, 64:72]], axis=1)
    kk = lax.broadcasted_iota(jnp.int32, (16, DIM), 0)
    cc = lax.broadcasted_iota(jnp.int32, (16, DIM), 1)
    c16 = cc // DK
    sel = (kk == c16 + 4 * (c16 >= 4)).astype(jnp.float32)
    den128 = lax.dot_general(dcat, sel, (((1,), (0,)), ((), ())))
    attn = nfull / (den128 + 1e-12)
    o_ref[...] = lax.dot_general(
        attn, wo_ref[...], (((1,), (1,)), ((), ()))) + bo_ref[...]


def _finalize(num_p, wo, bo):
    bn = 1000
    return pl.pallas_call(
        _fin_body,
        grid=(N // bn,),
        in_specs=[
            pl.BlockSpec((NC, bn, 80), lambda i: (0, i, 0)),
            pl.BlockSpec((DIM, DIM), lambda i: (0, 0)),
            pl.BlockSpec((1, DIM), lambda i: (0, 0)),
        ],
        out_specs=pl.BlockSpec((bn, DIM), lambda i: (i, 0)),
        out_shape=jax.ShapeDtypeStruct((N, DIM), jnp.float32),
    )(num_p, wo, bo)


def kernel(x, edge_index, edge_attr, W_Q, b_Q, W_K, b_K, W_V, b_V, W_O, b_O,
           eb_W1, eb_b1, eb_W2, eb_b2):
    ei = edge_index.astype(jnp.int32)
    qt, kt, vt = _tables(x, W_Q, b_Q[None, :], W_K, b_K[None, :], W_V, b_V[None, :])
    w2p = jnp.zeros((16, ED), jnp.float32).at[:H].set(eb_W2)
    b2p = jnp.zeros((16,), jnp.float32).at[:H].set(eb_b2)
    ebias = _edge_bias(edge_attr, eb_W1, eb_b1[None, :], w2p, b2p[None, :])
    ebias = ebias.reshape(E * 16 // 128, 128)
    num_p = _sc_attn(qt, kt, vt, ei[0], ei[1], ebias)
    return _finalize(num_p, W_O, b_O[None, :])
